# Initial kernel scaffold; baseline (speedup 1.0000x reference)
import jax
import jax.numpy as jnp
from jax.experimental import pallas as pl


def kernel(x, token_table, pos_table):
    raise NotImplementedError("write your pallas kernel here")


# speedup vs baseline: 1.2259x; 1.2259x over previous
import functools

import jax
import jax.numpy as jnp
from jax import lax
from jax.experimental import pallas as pl
from jax.experimental.pallas import tpu as pltpu
from jax.experimental.pallas import tpu_sc as plsc

BATCH = 4096
MAXLEN = 200
EMBED = 32
HALF = MAXLEN // 2

_info = plsc.get_sparse_core_info()
NC = _info.num_cores
NS = _info.num_subcores
NW = NC * NS
SEQ_PER_W = BATCH // NW


def _body(x_ref, tok_ref, pos_ref, out_ref, idx_v, pos_v, gbuf, obuf, sem_g, sem_o):
    wid = lax.axis_index("s") * NC + lax.axis_index("c")

    pltpu.sync_copy(x_ref.at[wid], idx_v)
    pltpu.sync_copy(pos_ref, pos_v)

    def step(i, carry):
        g0 = pltpu.make_async_copy(
            tok_ref.at[idx_v.at[2 * i]], gbuf.at[pl.ds(0, HALF)], sem_g)
        g1 = pltpu.make_async_copy(
            tok_ref.at[idx_v.at[2 * i + 1]], gbuf.at[pl.ds(HALF, HALF)], sem_g)
        g0.start()
        g1.start()
        g0.wait()
        g1.wait()

        for r in range(MAXLEN):
            for h in range(2):
                sl = pl.ds(h * 16, 16)
                obuf[r, sl] = gbuf[r, sl] + pos_v[r, sl]

        o = pltpu.make_async_copy(obuf, out_ref.at[wid * SEQ_PER_W + i], sem_o)
        o.start()
        o.wait()
        return carry

    lax.fori_loop(0, SEQ_PER_W, step, 0)


@functools.partial(jax.jit, static_argnums=())
def _embed(x3, token_table, pos_table):
    mesh = plsc.VectorSubcoreMesh(core_axis_name="c", subcore_axis_name="s")
    return pl.kernel(
        _body,
        mesh=mesh,
        compiler_params=pltpu.CompilerParams(use_tc_tiling_on_sc=False),
        out_type=jax.ShapeDtypeStruct((BATCH, MAXLEN, EMBED), jnp.float32),
        scratch_types=[
            pltpu.VMEM((2 * SEQ_PER_W, HALF), jnp.int32),
            pltpu.VMEM((MAXLEN, EMBED), jnp.float32),
            pltpu.VMEM((MAXLEN, EMBED), jnp.float32),
            pltpu.VMEM((MAXLEN, EMBED), jnp.float32),
            pltpu.SemaphoreType.DMA,
            pltpu.SemaphoreType.DMA,
        ],
    )(x3, token_table, pos_table)


def kernel(x, token_table, pos_table):
    x3 = x.astype(jnp.int32).reshape(NW, 2 * SEQ_PER_W, HALF)
    return _embed(x3, token_table, pos_table)
